# Initial kernel scaffold; baseline (speedup 1.0000x reference)
import functools

import jax
import jax.numpy as jnp
from jax import lax
from jax.experimental import pallas as pl
from jax.experimental.pallas import tpu as pltpu
from jax.experimental.pallas import tpu_sc as plsc

NUM_CORES = 2
NUM_SUBCORES = 16
LANES = 16
NW = NUM_CORES * NUM_SUBCORES

BATCH_TOTAL = 16384
B_PER_W = BATCH_TOTAL // NW
CHUNK = 128
N_CHUNKS = B_PER_W // CHUNK
GROUPS = CHUNK // LANES
EMBED = 128


def _tec_kernel(h_hbm, r_hbm, t_hbm, ent_hbm, rel_hbm, out_hbm,
                h_idx, r_idx, t_idx, h_buf, r_buf, t_buf, out_v, sem):
    wid = lax.axis_index("s") * NUM_CORES + lax.axis_index("c")

    pltpu.sync_copy(h_hbm.at[wid], h_idx)
    pltpu.sync_copy(r_hbm.at[wid], r_idx)
    pltpu.sync_copy(t_hbm.at[wid], t_idx)

    row_base = [lax.iota(jnp.int32, (LANES,)) + g * LANES for g in range(GROUPS)]

    for c in range(N_CHUNKS):
        cp_h = pltpu.async_copy(ent_hbm.at[h_idx.at[c]], h_buf, sem)
        cp_r = pltpu.async_copy(rel_hbm.at[r_idx.at[c]], r_buf, sem)
        cp_t = pltpu.async_copy(ent_hbm.at[t_idx.at[c]], t_buf, sem)
        cp_h.wait()
        cp_r.wait()
        cp_t.wait()

        def body(j, accs):
            col = jnp.full((LANES,), j, dtype=jnp.int32)
            new = []
            for g in range(GROUPS):
                hv = plsc.load_gather(h_buf, [row_base[g], col])
                rv = plsc.load_gather(r_buf, [row_base[g], col])
                tv = plsc.load_gather(t_buf, [row_base[g], col])
                new.append(accs[g] + jnp.abs(hv + rv - tv))
            return tuple(new)

        accs = lax.fori_loop(
            0, EMBED, body,
            tuple(jnp.zeros((LANES,), jnp.float32) for _ in range(GROUPS)))
        for g in range(GROUPS):
            out_v[pl.ds(c * CHUNK + g * LANES, LANES)] = -accs[g]

    pltpu.sync_copy(out_v, out_hbm.at[wid])


@jax.jit
def _transe_sc(h, r, t, entity_embeddings, relation_embeddings):
    mesh = plsc.VectorSubcoreMesh(core_axis_name="c", subcore_axis_name="s")
    kfn = functools.partial(
        pl.kernel,
        out_type=jax.ShapeDtypeStruct((NW, B_PER_W), jnp.float32),
        mesh=mesh,
        scratch_types=[
            pltpu.VMEM((N_CHUNKS, CHUNK), jnp.int32),
            pltpu.VMEM((N_CHUNKS, CHUNK), jnp.int32),
            pltpu.VMEM((N_CHUNKS, CHUNK), jnp.int32),
            pltpu.VMEM((CHUNK, EMBED), jnp.float32),
            pltpu.VMEM((CHUNK, EMBED), jnp.float32),
            pltpu.VMEM((CHUNK, EMBED), jnp.float32),
            pltpu.VMEM((B_PER_W,), jnp.float32),
            pltpu.SemaphoreType.DMA,
        ],
    )(_tec_kernel)
    h2 = h.astype(jnp.int32).reshape(NW, N_CHUNKS, CHUNK)
    r2 = r.astype(jnp.int32).reshape(NW, N_CHUNKS, CHUNK)
    t2 = t.astype(jnp.int32).reshape(NW, N_CHUNKS, CHUNK)
    out = kfn(h2, r2, t2, entity_embeddings, relation_embeddings)
    return out.reshape(BATCH_TOTAL)


def kernel(h, r, t, entity_embeddings, relation_embeddings):
    return _transe_sc(h, r, t, entity_embeddings, relation_embeddings)


# speedup vs baseline: 1.3447x; 1.3447x over previous
import functools

import jax
import jax.numpy as jnp
from jax import lax
from jax.experimental import pallas as pl
from jax.experimental.pallas import tpu as pltpu
from jax.experimental.pallas import tpu_sc as plsc

NUM_CORES = 2
NUM_SUBCORES = 16
LANES = 16
NW = NUM_CORES * NUM_SUBCORES

BATCH_TOTAL = 16384
B_PER_W = BATCH_TOTAL // NW
CHUNK = 128
N_CHUNKS = B_PER_W // CHUNK
GROUPS = CHUNK // LANES
EMBED = 128


def _tec_kernel(h_hbm, r_hbm, t_hbm, ent_hbm, rel_hbm, out_hbm,
                h_idx, r_idx, t_idx, h_buf, r_buf, t_buf, out_v, sem):
    wid = lax.axis_index("s") * NUM_CORES + lax.axis_index("c")

    pltpu.sync_copy(h_hbm.at[wid], h_idx)
    pltpu.sync_copy(r_hbm.at[wid], r_idx)
    pltpu.sync_copy(t_hbm.at[wid], t_idx)

    lane = lax.iota(jnp.int32, LANES)

    for c in range(N_CHUNKS):
        cp_h = pltpu.async_copy(ent_hbm.at[h_idx.at[c]], h_buf, sem)
        cp_r = pltpu.async_copy(rel_hbm.at[r_idx.at[c]], r_buf, sem)
        cp_t = pltpu.async_copy(ent_hbm.at[t_idx.at[c]], t_buf, sem)
        cp_h.wait()
        cp_r.wait()
        cp_t.wait()

        def group_body(g, _):
            acc = jnp.zeros((LANES,), jnp.float32)
            for i in range(LANES):
                row = g * LANES + i
                parts = []
                for s in range(EMBED // LANES):
                    sl = pl.ds(s * LANES, LANES)
                    hv = h_buf[row, sl]
                    rv = r_buf[row, sl]
                    tv = t_buf[row, sl]
                    parts.append(jnp.abs(hv + rv - tv))
                while len(parts) > 1:
                    parts = [a + b for a, b in zip(parts[::2], parts[1::2])]
                rowsum = lax.reduce_sum_p.bind(parts[0], axes=(0,))
                acc = jnp.where(lane == i, -rowsum, acc)
            out_v[pl.ds(c * CHUNK + g * LANES, LANES)] = acc
            return 0

        lax.fori_loop(0, GROUPS, group_body, 0)

    pltpu.sync_copy(out_v, out_hbm.at[wid])


@jax.jit
def _transe_sc(h, r, t, entity_embeddings, relation_embeddings):
    mesh = plsc.VectorSubcoreMesh(core_axis_name="c", subcore_axis_name="s")
    kfn = functools.partial(
        pl.kernel,
        out_type=jax.ShapeDtypeStruct((NW, B_PER_W), jnp.float32),
        mesh=mesh,
        compiler_params=pltpu.CompilerParams(needs_layout_passes=False),
        scratch_types=[
            pltpu.VMEM((N_CHUNKS, CHUNK), jnp.int32),
            pltpu.VMEM((N_CHUNKS, CHUNK), jnp.int32),
            pltpu.VMEM((N_CHUNKS, CHUNK), jnp.int32),
            pltpu.VMEM((CHUNK, EMBED), jnp.float32),
            pltpu.VMEM((CHUNK, EMBED), jnp.float32),
            pltpu.VMEM((CHUNK, EMBED), jnp.float32),
            pltpu.VMEM((B_PER_W,), jnp.float32),
            pltpu.SemaphoreType.DMA,
        ],
    )(_tec_kernel)
    h2 = h.astype(jnp.int32).reshape(NW, N_CHUNKS, CHUNK)
    r2 = r.astype(jnp.int32).reshape(NW, N_CHUNKS, CHUNK)
    t2 = t.astype(jnp.int32).reshape(NW, N_CHUNKS, CHUNK)
    out = kfn(h2, r2, t2, entity_embeddings, relation_embeddings)
    return out.reshape(BATCH_TOTAL)


def kernel(h, r, t, entity_embeddings, relation_embeddings):
    return _transe_sc(h, r, t, entity_embeddings, relation_embeddings)


# speedup vs baseline: 1.9189x; 1.4270x over previous
import functools

import jax
import jax.numpy as jnp
from jax import lax
from jax.experimental import pallas as pl
from jax.experimental.pallas import tpu as pltpu
from jax.experimental.pallas import tpu_sc as plsc

NUM_CORES = 2
NUM_SUBCORES = 16
LANES = 16
NW = NUM_CORES * NUM_SUBCORES

BATCH_TOTAL = 16384
B_PER_W = BATCH_TOTAL // NW
CHUNK = 128
N_CHUNKS = B_PER_W // CHUNK
GROUPS = CHUNK // LANES
EMBED = 128


def _tec_kernel(h_hbm, r_hbm, t_hbm, ent_hbm, rel_hbm, out_hbm,
                h_idx, r_idx, t_idx,
                h_buf0, r_buf0, t_buf0, h_buf1, r_buf1, t_buf1,
                out_v, sem0, sem1):
    wid = lax.axis_index("s") * NUM_CORES + lax.axis_index("c")

    pltpu.sync_copy(h_hbm.at[wid], h_idx)
    pltpu.sync_copy(r_hbm.at[wid], r_idx)
    pltpu.sync_copy(t_hbm.at[wid], t_idx)

    bufs = ((h_buf0, r_buf0, t_buf0), (h_buf1, r_buf1, t_buf1))
    sems = (sem0, sem1)

    def fire(c):
        hb, rb, tb = bufs[c & 1]
        sem = sems[c & 1]
        return (pltpu.async_copy(ent_hbm.at[h_idx.at[c]], hb, sem),
                pltpu.async_copy(rel_hbm.at[r_idx.at[c]], rb, sem),
                pltpu.async_copy(ent_hbm.at[t_idx.at[c]], tb, sem))

    cps = fire(0)
    for c in range(N_CHUNKS):
        for cp in cps:
            cp.wait()
        if c + 1 < N_CHUNKS:
            cps = fire(c + 1)
        hb, rb, tb = bufs[c & 1]

        lane = lax.iota(jnp.int32, LANES)

        @plsc.parallel_loop(0, GROUPS, step=1)
        def group_body(g):
            masked = []
            for i in range(LANES):
                row = g * LANES + i
                parts = []
                for s in range(EMBED // LANES):
                    sl = pl.ds(s * LANES, LANES)
                    parts.append(jnp.abs(hb[row, sl] + rb[row, sl] - tb[row, sl]))
                while len(parts) > 1:
                    parts = [a + b for a, b in zip(parts[::2], parts[1::2])]
                rowsum = lax.reduce_sum_p.bind(parts[0], axes=(0,))
                masked.append(jnp.where(lane == i, -rowsum, 0.0))
            while len(masked) > 1:
                masked = [a + b for a, b in zip(masked[::2], masked[1::2])]
            out_v[pl.ds(c * CHUNK + g * LANES, LANES)] = masked[0]

    pltpu.sync_copy(out_v, out_hbm.at[wid])


@jax.jit
def _transe_sc(h, r, t, entity_embeddings, relation_embeddings):
    mesh = plsc.VectorSubcoreMesh(core_axis_name="c", subcore_axis_name="s")
    kfn = functools.partial(
        pl.kernel,
        out_type=jax.ShapeDtypeStruct((NW, B_PER_W), jnp.float32),
        mesh=mesh,
        compiler_params=pltpu.CompilerParams(needs_layout_passes=False),
        scratch_types=[
            pltpu.VMEM((N_CHUNKS, CHUNK), jnp.int32),
            pltpu.VMEM((N_CHUNKS, CHUNK), jnp.int32),
            pltpu.VMEM((N_CHUNKS, CHUNK), jnp.int32),
            pltpu.VMEM((CHUNK, EMBED), jnp.float32),
            pltpu.VMEM((CHUNK, EMBED), jnp.float32),
            pltpu.VMEM((CHUNK, EMBED), jnp.float32),
            pltpu.VMEM((CHUNK, EMBED), jnp.float32),
            pltpu.VMEM((CHUNK, EMBED), jnp.float32),
            pltpu.VMEM((CHUNK, EMBED), jnp.float32),
            pltpu.VMEM((B_PER_W,), jnp.float32),
            pltpu.SemaphoreType.DMA,
            pltpu.SemaphoreType.DMA,
        ],
    )(_tec_kernel)
    h2 = h.astype(jnp.int32).reshape(NW, N_CHUNKS, CHUNK)
    r2 = r.astype(jnp.int32).reshape(NW, N_CHUNKS, CHUNK)
    t2 = t.astype(jnp.int32).reshape(NW, N_CHUNKS, CHUNK)
    out = kfn(h2, r2, t2, entity_embeddings, relation_embeddings)
    return out.reshape(BATCH_TOTAL)


def kernel(h, r, t, entity_embeddings, relation_embeddings):
    return _transe_sc(h, r, t, entity_embeddings, relation_embeddings)
